# Initial kernel scaffold; baseline (speedup 1.0000x reference)
import jax
import jax.numpy as jnp
from jax.experimental import pallas as pl


def kernel(boxes, scores, labels):
    raise NotImplementedError("write your pallas kernel here")


# speedup vs baseline: 95.6348x; 95.6348x over previous
import jax
import jax.numpy as jnp
from jax.experimental import pallas as pl
from jax.experimental.pallas import tpu as pltpu

_N = 20000
_B = 512
_NB = 40
_NPAD = _B * _NB
_SCORE_THRESH = 0.2
_NMS_THRESH = 0.5


def _iou_gt(cx1, cy1, cx2, cy2, ca, rx1, ry1, rx2, ry2, ra):
    xx1 = jnp.maximum(cx1, rx1)
    yy1 = jnp.maximum(cy1, ry1)
    xx2 = jnp.minimum(cx2, rx2)
    yy2 = jnp.minimum(cy2, ry2)
    inter = jnp.maximum(xx2 - xx1, 0.0) * jnp.maximum(yy2 - yy1, 0.0)
    iou = inter / (ca + ra - inter + 1e-9)
    return (iou > _NMS_THRESH).astype(jnp.float32)


def _nms_body(x1r, y1r, x2r, y2r, ar, vr, cpack, keep_r, keep_c):
    keep_r[...] = vr[...]
    keep_c[...] = cpack[:, :, 5:6]

    rows_i = jax.lax.broadcasted_iota(jnp.int32, (_B, _B), 0)
    cols_i = jax.lax.broadcasted_iota(jnp.int32, (_B, _B), 1)
    upper = (rows_i < cols_i).astype(jnp.float32)
    lower = (cols_i < rows_i).astype(jnp.float32)

    def load_col(k):
        blk = cpack[pl.ds(k, 1), :, :].reshape(_B, 6)
        return (blk[:, 0:1], blk[:, 1:2], blk[:, 2:3], blk[:, 3:4], blk[:, 4:5])

    def load_row(k):
        sl = (pl.ds(k, 1), slice(None))
        return (x1r[sl], y1r[sl], x2r[sl], y2r[sl], ar[sl])

    def outer(k, carry):
        ck = load_col(k)
        rk = load_row(k)
        p = _iou_gt(*ck, *rk)
        mu = p * upper
        ml = p * lower
        m_r = keep_r[pl.ds(k, 1), :]
        m_c = keep_c[pl.ds(k, 1), :, :].reshape(_B, 1)

        def cond(st):
            return st[2]

        def body(st):
            c, _, _ = st
            supp_r = jnp.max(mu * c, axis=0, keepdims=True)
            r2 = m_r * (1.0 - supp_r)
            supp_c = jnp.max(ml * r2, axis=1, keepdims=True)
            c2 = m_c * (1.0 - supp_c)
            return (c2, r2, jnp.any(c2 != c))

        c_fin, r_fin, _ = jax.lax.while_loop(
            cond, body, (m_c, m_r, jnp.bool_(True))
        )
        keep_r[pl.ds(k, 1), :] = r_fin
        keep_c[pl.ds(k, 1), :, :] = c_fin.reshape(1, _B, 1)

        def inner(j, carry2):
            rj = load_row(j)
            p1 = _iou_gt(*ck, *rj)
            supp_r = jnp.max(p1 * c_fin, axis=0, keepdims=True)
            keep_r[pl.ds(j, 1), :] = keep_r[pl.ds(j, 1), :] * (1.0 - supp_r)
            cj = load_col(j)
            p2 = _iou_gt(*cj, *rk)
            supp_c = jnp.max(p2 * r_fin, axis=1, keepdims=True)
            keep_c[pl.ds(j, 1), :, :] = (
                keep_c[pl.ds(j, 1), :, :] * (1.0 - supp_c.reshape(1, _B, 1))
            )
            return carry2

        jax.lax.fori_loop(k + 1, _NB, inner, 0)
        return carry

    jax.lax.fori_loop(0, _NB, outer, 0)


def kernel(boxes, scores, labels):
    valid = scores > _SCORE_THRESH
    masked = jnp.where(valid, scores, -1.0)
    order = jnp.argsort(-masked)
    b = boxes[order]
    s = scores[order]
    l = labels[order]
    v = valid[order]
    max_coord = jnp.max(boxes) + 1.0
    bo = b + (l.astype(boxes.dtype) * max_coord)[:, None]
    areas = (bo[:, 2] - bo[:, 0]) * (bo[:, 3] - bo[:, 1])
    vf = v.astype(jnp.float32)

    feat = jnp.concatenate([bo, areas[:, None], vf[:, None]], axis=1)
    featp = jnp.pad(feat, ((0, _NPAD - _N), (0, 0)))
    rform = featp.reshape(_NB, _B, 6)
    r_args = [rform[:, :, c] for c in range(6)]

    keep = pl.pallas_call(
        _nms_body,
        out_shape=jax.ShapeDtypeStruct((_NB, _B), jnp.float32),
        in_specs=[pl.BlockSpec(memory_space=pltpu.VMEM)] * 7,
        out_specs=pl.BlockSpec(memory_space=pltpu.VMEM),
        scratch_shapes=[pltpu.VMEM((_NB, _B, 1), jnp.float32)],
    )(*r_args, rform)

    km = keep.reshape(_NPAD)[:_N]
    return jnp.concatenate([b * km[:, None], (s * km)[:, None]], axis=1)


# speedup vs baseline: 317.4661x; 3.3196x over previous
import jax
import jax.numpy as jnp
from jax.experimental import pallas as pl
from jax.experimental.pallas import tpu as pltpu

_N = 20000
_B = 512
_NB = 40
_NPAD = _B * _NB
_SCORE_THRESH = 0.2
_NMS_THRESH = 0.5


def _iou_gt(cx1, cy1, cx2, cy2, ca, rx1, ry1, rx2, ry2, ra):
    xx1 = jnp.maximum(cx1, rx1)
    yy1 = jnp.maximum(cy1, ry1)
    xx2 = jnp.minimum(cx2, rx2)
    yy2 = jnp.minimum(cy2, ry2)
    inter = jnp.maximum(xx2 - xx1, 0.0) * jnp.maximum(yy2 - yy1, 0.0)
    iou = inter / (ca + ra - inter + 1e-9)
    return (iou > _NMS_THRESH).astype(jnp.float32)


def _nms_body(jhi, x1r, y1r, x2r, y2r, ar, vr, cpack, keep_r, keep_c):
    keep_r[...] = vr[...]
    keep_c[...] = cpack[:, :, 5:6]

    rows_i = jax.lax.broadcasted_iota(jnp.int32, (_B, _B), 0)
    cols_i = jax.lax.broadcasted_iota(jnp.int32, (_B, _B), 1)
    upper = (rows_i < cols_i).astype(jnp.float32)
    lower = (cols_i < rows_i).astype(jnp.float32)

    def load_col(k):
        blk = cpack[pl.ds(k, 1), :, :].reshape(_B, 6)
        return (blk[:, 0:1], blk[:, 1:2], blk[:, 2:3], blk[:, 3:4], blk[:, 4:5])

    def load_row(k):
        sl = (pl.ds(k, 1), slice(None))
        return (x1r[sl], y1r[sl], x2r[sl], y2r[sl], ar[sl])

    def outer(k, carry):
        ck = load_col(k)
        rk = load_row(k)
        p = _iou_gt(*ck, *rk)
        mu = p * upper
        ml = p * lower
        m_r = keep_r[pl.ds(k, 1), :]
        m_c = keep_c[pl.ds(k, 1), :, :].reshape(_B, 1)

        def cond(st):
            return st[2]

        def body(st):
            c, _, _ = st
            supp_r = jnp.max(mu * c, axis=0, keepdims=True)
            r2 = m_r * (1.0 - supp_r)
            supp_c = jnp.max(ml * r2, axis=1, keepdims=True)
            c2 = m_c * (1.0 - supp_c)
            return (c2, r2, jnp.any(c2 != c))

        c_fin, r_fin, _ = jax.lax.while_loop(
            cond, body, (m_c, m_r, jnp.bool_(True))
        )
        keep_r[pl.ds(k, 1), :] = r_fin
        keep_c[pl.ds(k, 1), :, :] = c_fin.reshape(1, _B, 1)

        def inner(j, carry2):
            rj = load_row(j)
            p1 = _iou_gt(*ck, *rj)
            supp_r = jnp.max(p1 * c_fin, axis=0, keepdims=True)
            keep_r[pl.ds(j, 1), :] = keep_r[pl.ds(j, 1), :] * (1.0 - supp_r)
            cj = load_col(j)
            p2 = _iou_gt(*cj, *rk)
            supp_c = jnp.max(p2 * r_fin, axis=1, keepdims=True)
            keep_c[pl.ds(j, 1), :, :] = (
                keep_c[pl.ds(j, 1), :, :] * (1.0 - supp_c.reshape(1, _B, 1))
            )
            return carry2

        jax.lax.fori_loop(k + 1, jnp.maximum(jhi[k], k + 1), inner, 0)
        return carry

    jax.lax.fori_loop(0, _NB, outer, 0)


def kernel(boxes, scores, labels):
    valid = scores > _SCORE_THRESH
    masked = jnp.where(valid, scores, -1.0)
    order = jnp.argsort(-masked)
    b = boxes[order]
    s = scores[order]

    perm2 = jnp.argsort(labels[order], stable=True)
    ord2 = order[perm2]
    b2 = boxes[ord2]
    l2 = labels[ord2]
    v2 = valid[ord2]

    max_coord = jnp.max(boxes) + 1.0
    bo = b2 + (l2.astype(boxes.dtype) * max_coord)[:, None]
    areas = (bo[:, 2] - bo[:, 0]) * (bo[:, 3] - bo[:, 1])
    vf = v2.astype(jnp.float32)

    feat = jnp.concatenate([bo, areas[:, None], vf[:, None]], axis=1)
    featp = jnp.pad(feat, ((0, _NPAD - _N), (0, 0)))
    rform = featp.reshape(_NB, _B, 6)
    r_args = [rform[:, :, c] for c in range(6)]

    l2p = jnp.pad(l2, (0, _NPAD - _N), constant_values=jnp.int32(2**30))
    lblk = l2p.reshape(_NB, _B)
    bmin = lblk.min(axis=1)
    bmax = lblk.max(axis=1)
    jhi = jnp.searchsorted(bmin, bmax, side="right").astype(jnp.int32)

    keep2 = pl.pallas_call(
        _nms_body,
        out_shape=jax.ShapeDtypeStruct((_NB, _B), jnp.float32),
        in_specs=[pl.BlockSpec(memory_space=pltpu.SMEM)]
        + [pl.BlockSpec(memory_space=pltpu.VMEM)] * 7,
        out_specs=pl.BlockSpec(memory_space=pltpu.VMEM),
        scratch_shapes=[pltpu.VMEM((_NB, _B, 1), jnp.float32)],
    )(jhi, *r_args, rform)

    km = jnp.zeros((_N,), jnp.float32).at[perm2].set(
        keep2.reshape(_NPAD)[:_N], unique_indices=True
    )
    return jnp.concatenate([b * km[:, None], (s * km)[:, None]], axis=1)
